# Initial kernel scaffold; baseline (speedup 1.0000x reference)
import jax
import jax.numpy as jnp
from jax.experimental import pallas as pl


def kernel(x, seg, word_emb, seg_emb, gamma, beta):
    raise NotImplementedError("write your pallas kernel here")


# speedup vs baseline: 1.4820x; 1.4820x over previous
import functools

import jax
import jax.numpy as jnp
from jax import lax
from jax.experimental import pallas as pl
from jax.experimental.pallas import tpu as pltpu
from jax.experimental.pallas import tpu_sc as plsc

D_MODEL = 224
NV = 1955
NSEG = 2
EPS = 1e-5
LANES = 16
NUM_CORES = 2
NUM_SUBCORES = 16
NW = NUM_CORES * NUM_SUBCORES
NJ = D_MODEL // LANES


def _comb_body(word_ref, seg_ref, out_ref):
    out_ref[0] = word_ref[...] + seg_ref[0:1, :]
    out_ref[1] = word_ref[...] + seg_ref[1:2, :]


def _build_comb(word_emb, seg_emb):
    comb = pl.pallas_call(
        _comb_body,
        out_shape=jax.ShapeDtypeStruct((NSEG, NV, D_MODEL), jnp.float32),
    )(word_emb, seg_emb)
    return comb.reshape(NSEG * NV, D_MODEL)


_GDN = lax.GatherDimensionNumbers(
    offset_dims=(), collapsed_slice_dims=(0,), start_index_map=(0,))


def _shuffle(v, idx):
    return lax.gather(v, idx[:, None], _GDN, slice_sizes=(1,),
                      mode=lax.GatherScatterMode.PROMISE_IN_BOUNDS)


def _xlane_sum(v):
    idx = lax.iota(jnp.int32, LANES)
    for sh in (1, 2, 4, 8):
        v = v + _shuffle(v, idx ^ sh)
    return v


def _rsqrt(v):
    i = lax.bitcast_convert_type(v, jnp.int32)
    y = lax.bitcast_convert_type(0x5F3759DF - (i >> 1), jnp.float32)
    for _ in range(3):
        y = y * (1.5 - 0.5 * v * y * y)
    return y


def _make_sc_kernel(n_tok, k_chunk):
    per_w = n_tok // NW
    n_chunks = per_w // k_chunk
    mesh = plsc.VectorSubcoreMesh(
        core_axis_name="c", subcore_axis_name="s",
        num_cores=NUM_CORES, num_subcores=NUM_SUBCORES,
    )

    @functools.partial(
        pl.kernel,
        mesh=mesh,
        compiler_params=pltpu.CompilerParams(use_tc_tiling_on_sc=False),
        out_type=jax.ShapeDtypeStruct((n_tok, D_MODEL), jnp.float32),
        scratch_types=[
            pltpu.VMEM((k_chunk,), jnp.int32),
            pltpu.VMEM((k_chunk,), jnp.int32),
            pltpu.VMEM((k_chunk,), jnp.int32),
            pltpu.VMEM((k_chunk, D_MODEL), jnp.float32),
            pltpu.VMEM((D_MODEL,), jnp.float32),
            pltpu.VMEM((D_MODEL,), jnp.float32),
            pltpu.SemaphoreType.DMA,
        ],
    )
    def sc_kernel(comb_hbm, xf_hbm, segf_hbm, gamma_hbm, beta_hbm, out_hbm,
                  xv, sv, cv, rows, gv, bv, gsem):
        wid = lax.axis_index("s") * NUM_CORES + lax.axis_index("c")
        base = wid * per_w
        pltpu.sync_copy(gamma_hbm, gv)
        pltpu.sync_copy(beta_hbm, bv)

        def chunk_body(c, carry):
            off = base + c * k_chunk
            pltpu.sync_copy(xf_hbm.at[pl.ds(off, k_chunk)], xv)
            pltpu.sync_copy(segf_hbm.at[pl.ds(off, k_chunk)], sv)
            for k in range(k_chunk // LANES):
                sl = pl.ds(k * LANES, LANES)
                cv[sl] = sv[sl] * NV + xv[sl]
            pltpu.async_copy(comb_hbm.at[cv], rows, gsem).wait()

            def tok(i, tcarry):
                h = [rows[i, pl.ds(j * LANES, LANES)] for j in range(NJ)]
                svec = h[0]
                qvec = h[0] * h[0]
                for j in range(1, NJ):
                    svec = svec + h[j]
                    qvec = qvec + h[j] * h[j]
                sb = _xlane_sum(svec)
                qb = _xlane_sum(qvec)
                meanv = sb * (1.0 / D_MODEL)
                varv = qb * (1.0 / D_MODEL) - meanv * meanv + EPS
                rstd = _rsqrt(varv)
                for j in range(NJ):
                    sl = pl.ds(j * LANES, LANES)
                    rows[i, sl] = (h[j] - meanv) * (rstd * gv[sl]) + bv[sl]
                return tcarry

            lax.fori_loop(0, k_chunk, tok, 0)
            pltpu.sync_copy(rows, out_hbm.at[pl.ds(off, k_chunk)])
            return carry

        lax.fori_loop(0, n_chunks, chunk_body, 0)

    return sc_kernel


def kernel(x, seg, word_emb, seg_emb, gamma, beta):
    b, l = x.shape
    n_tok = b * l
    comb = _build_comb(word_emb, seg_emb)
    xf = x.reshape(n_tok)
    segf = seg.reshape(n_tok)
    sc = _make_sc_kernel(n_tok, 128)
    out = sc(comb, xf, segf, gamma, beta)
    return out.reshape(b, l, D_MODEL)


# speedup vs baseline: 1.6165x; 1.0908x over previous
import functools

import jax
import jax.numpy as jnp
from jax import lax
from jax.experimental import pallas as pl
from jax.experimental.pallas import tpu as pltpu
from jax.experimental.pallas import tpu_sc as plsc

D_MODEL = 224
NV = 1955
NSEG = 2
EPS = 1e-5
LANES = 16
NUM_CORES = 2
NUM_SUBCORES = 16
NW = NUM_CORES * NUM_SUBCORES
NJ = D_MODEL // LANES
UNROLL = 4


def _comb_body(word_ref, seg_ref, out_ref):
    out_ref[0] = word_ref[...] + seg_ref[0:1, :]
    out_ref[1] = word_ref[...] + seg_ref[1:2, :]


def _build_comb(word_emb, seg_emb):
    comb = pl.pallas_call(
        _comb_body,
        out_shape=jax.ShapeDtypeStruct((NSEG, NV, D_MODEL), jnp.float32),
    )(word_emb, seg_emb)
    return comb.reshape(NSEG * NV, D_MODEL)


_GDN = lax.GatherDimensionNumbers(
    offset_dims=(), collapsed_slice_dims=(0,), start_index_map=(0,))


def _shuffle(v, idx):
    return lax.gather(v, idx[:, None], _GDN, slice_sizes=(1,),
                      mode=lax.GatherScatterMode.PROMISE_IN_BOUNDS)


def _xlane_sum(v):
    idx = lax.iota(jnp.int32, LANES)
    for sh in (1, 2, 4, 8):
        v = v + _shuffle(v, idx ^ sh)
    return v


def _rsqrt(v):
    i = lax.bitcast_convert_type(v, jnp.int32)
    y = lax.bitcast_convert_type(0x5F3759DF - (i >> 1), jnp.float32)
    for _ in range(3):
        y = y * (1.5 - 0.5 * v * y * y)
    return y


def _ln_token(rows, i, meanr, gv, bv):
    h = [rows[i, pl.ds(j * LANES, LANES)] for j in range(NJ)]
    svec = h[0]
    qvec = h[0] * h[0]
    for j in range(1, NJ):
        svec = svec + h[j]
        qvec = qvec + h[j] * h[j]
    sb = _xlane_sum(svec)
    qb = _xlane_sum(qvec)
    meanv = sb * meanr
    varv = qb * meanr - meanv * meanv + EPS
    rstd = _rsqrt(varv)
    for j in range(NJ):
        sl = pl.ds(j * LANES, LANES)
        rows[i, sl] = (h[j] - meanv) * (rstd * gv[sl]) + bv[sl]


def _make_sc_kernel(n_tok, k_chunk):
    per_w = n_tok // NW
    n_chunks = per_w // k_chunk
    mesh = plsc.VectorSubcoreMesh(
        core_axis_name="c", subcore_axis_name="s",
        num_cores=NUM_CORES, num_subcores=NUM_SUBCORES,
    )

    @functools.partial(
        pl.kernel,
        mesh=mesh,
        compiler_params=pltpu.CompilerParams(use_tc_tiling_on_sc=False),
        out_type=jax.ShapeDtypeStruct((n_tok, D_MODEL), jnp.float32),
        scratch_types=[
            pltpu.VMEM((k_chunk,), jnp.int32),
            pltpu.VMEM((k_chunk,), jnp.int32),
            pltpu.VMEM((k_chunk,), jnp.int32),
            pltpu.VMEM((k_chunk, D_MODEL), jnp.float32),
            pltpu.VMEM((D_MODEL,), jnp.float32),
            pltpu.VMEM((D_MODEL,), jnp.float32),
            pltpu.SemaphoreType.DMA,
        ],
    )
    def sc_kernel(comb_hbm, xf_hbm, segf_hbm, gamma_hbm, beta_hbm, out_hbm,
                  xv, sv, cv, rows, gv, bv, gsem):
        wid = lax.axis_index("s") * NUM_CORES + lax.axis_index("c")
        base = wid * per_w
        pltpu.sync_copy(gamma_hbm, gv)
        pltpu.sync_copy(beta_hbm, bv)

        def chunk_body(c, carry):
            off = base + c * k_chunk
            pltpu.sync_copy(xf_hbm.at[pl.ds(off, k_chunk)], xv)
            pltpu.sync_copy(segf_hbm.at[pl.ds(off, k_chunk)], sv)
            for k in range(k_chunk // LANES):
                sl = pl.ds(k * LANES, LANES)
                cv[sl] = sv[sl] * NV + xv[sl]
            pltpu.async_copy(comb_hbm.at[cv], rows, gsem).wait()
            meanr = jnp.full((LANES,), 1.0 / D_MODEL, jnp.float32)

            def tok(i, tcarry):
                for u in range(UNROLL):
                    _ln_token(rows, i * UNROLL + u, meanr, gv, bv)
                return tcarry

            lax.fori_loop(0, k_chunk // UNROLL, tok, 0)
            pltpu.sync_copy(rows, out_hbm.at[pl.ds(off, k_chunk)])
            return carry

        lax.fori_loop(0, n_chunks, chunk_body, 0)

    return sc_kernel


def kernel(x, seg, word_emb, seg_emb, gamma, beta):
    b, l = x.shape
    n_tok = b * l
    comb = _build_comb(word_emb, seg_emb)
    xf = x.reshape(n_tok)
    segf = seg.reshape(n_tok)
    sc = _make_sc_kernel(n_tok, 128)
    out = sc(comb, xf, segf, gamma, beta)
    return out.reshape(b, l, D_MODEL)
